# Initial kernel scaffold; baseline (speedup 1.0000x reference)
import jax
import jax.numpy as jnp
from jax.experimental import pallas as pl


def kernel(vectors, x, V, senders, species, W1, W2a, W2b, W2c, Wlin):
    raise NotImplementedError("write your pallas kernel here")


# speedup vs baseline: 10.7492x; 10.7492x over previous
import functools

import jax
import jax.numpy as jnp
import numpy as np
from jax import lax
from jax.experimental import pallas as pl
from jax.experimental.pallas import tpu as pltpu

N_NODES = 50000
N_EDGES = 800000
MUL = 16
X_DIM = 64
HIDDEN = 64
EPS = 0.25
SQRT3 = 1.7320508075688772

BE = 2000


def _pass1_body(x_ref, vec_ref, w1_ref, msg0_ref, msg1_ref):
    x = x_ref[...]
    v = vec_ref[...]
    w = jnp.dot(x, w1_ref[...], preferred_element_type=jnp.float32) * (1.0 / 8.0)
    n = jnp.sqrt(jnp.sum(v * v, axis=1, keepdims=True))
    u = v / jnp.maximum(n, 1e-9)
    c0 = w * EPS
    c1 = w * (EPS * SQRT3 * u[:, 0:1])
    c2 = w * (EPS * SQRT3 * u[:, 1:2])
    c3 = w * (EPS * SQRT3 * u[:, 2:3])
    msg0_ref[...] = jnp.concatenate([c0, c1], axis=1)
    msg1_ref[...] = jnp.concatenate([c2, c3], axis=1)


def _pass1(x, vectors, W1):
    grid = N_EDGES // BE
    return pl.pallas_call(
        _pass1_body,
        grid=(grid,),
        in_specs=[
            pl.BlockSpec((BE, X_DIM), lambda i: (i, 0)),
            pl.BlockSpec((BE, 3), lambda i: (i, 0)),
            pl.BlockSpec((X_DIM, MUL), lambda i: (0, 0)),
        ],
        out_specs=[
            pl.BlockSpec((BE, 2 * MUL), lambda i: (i, 0)),
            pl.BlockSpec((BE, 2 * MUL), lambda i: (i, 0)),
        ],
        out_shape=[
            jax.ShapeDtypeStruct((N_EDGES, 2 * MUL), jnp.float32),
            jax.ShapeDtypeStruct((N_EDGES, 2 * MUL), jnp.float32),
        ],
    )(x, vectors, W1)


def _pass2_body(x_ref, v_ref, wy0_ref, wy1_ref, vec_ref, w2a_ref, w2b_ref,
                w2c_ref, wlin_ref, p64_ref, p48_ref, xout_ref, vout_ref):
    x = x_ref[...]
    vcm = jnp.dot(v_ref[...], p64_ref[...], preferred_element_type=jnp.float32)
    wy0 = wy0_ref[...]
    wy1 = wy1_ref[...]
    s_w = wy0[:, 0:MUL]
    vwx = wy0[:, MUL:2 * MUL]
    vwy = wy1[:, 0:MUL]
    vwz = wy1[:, MUL:2 * MUL]
    s_v = vcm[:, 0:MUL]
    vvx = vcm[:, MUL:2 * MUL]
    vvy = vcm[:, 2 * MUL:3 * MUL]
    vvz = vcm[:, 3 * MUL:4 * MUL]
    sc0 = s_w * s_v
    sc1 = (vwx * vvx + vwy * vvy + vwz * vvz) * (1.0 / SQRT3)
    w2a = w2a_ref[...]
    pre = (jnp.dot(x, w2a[0:X_DIM], preferred_element_type=jnp.float32)
           + jnp.dot(sc0, w2a[X_DIM:X_DIM + MUL], preferred_element_type=jnp.float32)
           + jnp.dot(sc1, w2a[X_DIM + MUL:X_DIM + 2 * MUL], preferred_element_type=jnp.float32))
    h = jax.nn.silu(pre * (1.0 / jnp.sqrt(jnp.float32(X_DIM + 2 * MUL))))
    h = jax.nn.silu(jnp.dot(h, w2b_ref[...], preferred_element_type=jnp.float32) * (1.0 / 8.0))
    h = jnp.dot(h, w2c_ref[...], preferred_element_type=jnp.float32) * (1.0 / 8.0)
    v = vec_ref[...]
    u = jnp.sqrt(jnp.sum(v * v, axis=1, keepdims=True))
    val = 1.0 - 28.0 * u**6 + 48.0 * u**7 - 21.0 * u**8
    env = jnp.where(u < 1.0, val, 0.0)
    xout_ref[...] = env * h
    wlin = wlin_ref[...]
    wl0 = wlin[0:MUL]
    wl1 = wlin[MUL:2 * MUL]
    s = 1.0 / jnp.sqrt(jnp.float32(2 * MUL))
    outs = []
    for vv_i, vw_i in ((vvx, vwx), (vvy, vwy), (vvz, vwz)):
        a = s_w * vv_i
        b = vw_i * s_v
        outs.append((jnp.dot(a, wl0, preferred_element_type=jnp.float32)
                     + jnp.dot(b, wl1, preferred_element_type=jnp.float32)) * s)
    vout = jnp.concatenate(outs, axis=1)
    vout_ref[...] = jnp.dot(vout, p48_ref[...], preferred_element_type=jnp.float32)


def _pass2(x, V, wy0, wy1, vectors, W2a, W2b, W2c, Wlin, P64, P48):
    grid = N_EDGES // BE
    return pl.pallas_call(
        _pass2_body,
        grid=(grid,),
        in_specs=[
            pl.BlockSpec((BE, X_DIM), lambda i: (i, 0)),
            pl.BlockSpec((BE, 4 * MUL), lambda i: (i, 0)),
            pl.BlockSpec((BE, 2 * MUL), lambda i: (i, 0)),
            pl.BlockSpec((BE, 2 * MUL), lambda i: (i, 0)),
            pl.BlockSpec((BE, 3), lambda i: (i, 0)),
            pl.BlockSpec((X_DIM + 2 * MUL, HIDDEN), lambda i: (0, 0)),
            pl.BlockSpec((HIDDEN, HIDDEN), lambda i: (0, 0)),
            pl.BlockSpec((HIDDEN, HIDDEN), lambda i: (0, 0)),
            pl.BlockSpec((2 * MUL, MUL), lambda i: (0, 0)),
            pl.BlockSpec((4 * MUL, 4 * MUL), lambda i: (0, 0)),
            pl.BlockSpec((3 * MUL, 3 * MUL), lambda i: (0, 0)),
        ],
        out_specs=[
            pl.BlockSpec((BE, HIDDEN), lambda i: (i, 0)),
            pl.BlockSpec((BE, 3 * MUL), lambda i: (i, 0)),
        ],
        out_shape=[
            jax.ShapeDtypeStruct((N_EDGES, HIDDEN), jnp.float32),
            jax.ShapeDtypeStruct((N_EDGES, 3 * MUL), jnp.float32),
        ],
    )(x, V, wy0, wy1, vectors, W2a, W2b, W2c, Wlin, P64, P48)


def _perm64():
    p = np.zeros((4 * MUL, 4 * MUL), np.float32)
    for c in range(MUL):
        for i in range(4):
            p[4 * c + i, MUL * i + c] = 1.0
    return jnp.asarray(p)


def _perm48():
    p = np.zeros((3 * MUL, 3 * MUL), np.float32)
    for i in range(3):
        for o in range(MUL):
            p[MUL * i + o, 3 * o + i] = 1.0
    return jnp.asarray(p)


def _segment_mid(msg0, msg1, senders):
    n0 = jax.ops.segment_sum(msg0, senders, num_segments=N_NODES)
    n1 = jax.ops.segment_sum(msg1, senders, num_segments=N_NODES)
    return n0[senders], n1[senders]


def kernel(vectors, x, V, senders, species, W1, W2a, W2b, W2c, Wlin):
    msg0, msg1 = _pass1(x, vectors, W1)
    wy0, wy1 = _segment_mid(msg0, msg1, senders)
    x_out, v_out = _pass2(x, V, wy0, wy1, vectors, W2a, W2b, W2c, Wlin,
                          _perm64(), _perm48())
    return x_out, v_out


# speedup vs baseline: 18.7048x; 1.7401x over previous
import functools

import jax
import jax.numpy as jnp
import numpy as np
from jax import lax
from jax.experimental import pallas as pl
from jax.experimental.pallas import tpu as pltpu
from jax.experimental.pallas import tpu_sc as plsc

N_NODES = 50000
N_EDGES = 800000
MUL = 16
X_DIM = 64
HIDDEN = 64
EPS = 0.25
SQRT3 = 1.7320508075688772

BE = 2000


def _pass1_body(x_ref, vec_ref, w1_ref, msg_ref):
    x = x_ref[...]
    v = vec_ref[...]
    w = jnp.dot(x, w1_ref[...], preferred_element_type=jnp.float32) * (1.0 / 8.0)
    n = jnp.sqrt(jnp.sum(v * v, axis=1, keepdims=True))
    u = v / jnp.maximum(n, 1e-9)
    c0 = w * EPS
    c1 = w * (EPS * SQRT3 * u[:, 0:1])
    c2 = w * (EPS * SQRT3 * u[:, 1:2])
    c3 = w * (EPS * SQRT3 * u[:, 2:3])
    msg_ref[0] = jnp.concatenate([c0, c1], axis=1)
    msg_ref[1] = jnp.concatenate([c2, c3], axis=1)


def _pass1(x, vectors, W1):
    grid = N_EDGES // BE
    return pl.pallas_call(
        _pass1_body,
        grid=(grid,),
        in_specs=[
            pl.BlockSpec((BE, X_DIM), lambda i: (i, 0)),
            pl.BlockSpec((BE, 3), lambda i: (i, 0)),
            pl.BlockSpec((X_DIM, MUL), lambda i: (0, 0)),
        ],
        out_specs=[
            pl.BlockSpec((2, BE, 2 * MUL), lambda i: (0, i, 0)),
        ],
        out_shape=[
            jax.ShapeDtypeStruct((2, N_EDGES, 2 * MUL), jnp.float32),
        ],
    )(x, vectors, W1)[0]


def _pass2_body(x_ref, v_ref, wy_ref, vec_ref, w2a_ref, w2b_ref,
                w2c_ref, wlin_ref, p64_ref, p48_ref, xout_ref, vout_ref):
    x = x_ref[...]
    vcm = jnp.dot(v_ref[...], p64_ref[...], preferred_element_type=jnp.float32)
    wy0 = wy_ref[0]
    wy1 = wy_ref[1]
    s_w = wy0[:, 0:MUL]
    vwx = wy0[:, MUL:2 * MUL]
    vwy = wy1[:, 0:MUL]
    vwz = wy1[:, MUL:2 * MUL]
    s_v = vcm[:, 0:MUL]
    vvx = vcm[:, MUL:2 * MUL]
    vvy = vcm[:, 2 * MUL:3 * MUL]
    vvz = vcm[:, 3 * MUL:4 * MUL]
    sc0 = s_w * s_v
    sc1 = (vwx * vvx + vwy * vvy + vwz * vvz) * (1.0 / SQRT3)
    w2a = w2a_ref[...]
    pre = (jnp.dot(x, w2a[0:X_DIM], preferred_element_type=jnp.float32)
           + jnp.dot(sc0, w2a[X_DIM:X_DIM + MUL], preferred_element_type=jnp.float32)
           + jnp.dot(sc1, w2a[X_DIM + MUL:X_DIM + 2 * MUL], preferred_element_type=jnp.float32))
    h = jax.nn.silu(pre * (1.0 / jnp.sqrt(jnp.float32(X_DIM + 2 * MUL))))
    h = jax.nn.silu(jnp.dot(h, w2b_ref[...], preferred_element_type=jnp.float32) * (1.0 / 8.0))
    h = jnp.dot(h, w2c_ref[...], preferred_element_type=jnp.float32) * (1.0 / 8.0)
    v = vec_ref[...]
    u = jnp.sqrt(jnp.sum(v * v, axis=1, keepdims=True))
    val = 1.0 - 28.0 * u**6 + 48.0 * u**7 - 21.0 * u**8
    env = jnp.where(u < 1.0, val, 0.0)
    xout_ref[...] = env * h
    wlin = wlin_ref[...]
    wl0 = wlin[0:MUL]
    wl1 = wlin[MUL:2 * MUL]
    s = 1.0 / jnp.sqrt(jnp.float32(2 * MUL))
    outs = []
    for vv_i, vw_i in ((vvx, vwx), (vvy, vwy), (vvz, vwz)):
        a = s_w * vv_i
        b = vw_i * s_v
        outs.append((jnp.dot(a, wl0, preferred_element_type=jnp.float32)
                     + jnp.dot(b, wl1, preferred_element_type=jnp.float32)) * s)
    vout = jnp.concatenate(outs, axis=1)
    vout_ref[...] = jnp.dot(vout, p48_ref[...], preferred_element_type=jnp.float32)


def _pass2(x, V, wy, vectors, W2a, W2b, W2c, Wlin, P64, P48):
    grid = N_EDGES // BE
    return pl.pallas_call(
        _pass2_body,
        grid=(grid,),
        in_specs=[
            pl.BlockSpec((BE, X_DIM), lambda i: (i, 0)),
            pl.BlockSpec((BE, 4 * MUL), lambda i: (i, 0)),
            pl.BlockSpec((2, BE, 2 * MUL), lambda i: (0, i, 0)),
            pl.BlockSpec((BE, 3), lambda i: (i, 0)),
            pl.BlockSpec((X_DIM + 2 * MUL, HIDDEN), lambda i: (0, 0)),
            pl.BlockSpec((HIDDEN, HIDDEN), lambda i: (0, 0)),
            pl.BlockSpec((HIDDEN, HIDDEN), lambda i: (0, 0)),
            pl.BlockSpec((2 * MUL, MUL), lambda i: (0, 0)),
            pl.BlockSpec((4 * MUL, 4 * MUL), lambda i: (0, 0)),
            pl.BlockSpec((3 * MUL, 3 * MUL), lambda i: (0, 0)),
        ],
        out_specs=[
            pl.BlockSpec((BE, HIDDEN), lambda i: (i, 0)),
            pl.BlockSpec((BE, 3 * MUL), lambda i: (i, 0)),
        ],
        out_shape=[
            jax.ShapeDtypeStruct((N_EDGES, HIDDEN), jnp.float32),
            jax.ShapeDtypeStruct((N_EDGES, 3 * MUL), jnp.float32),
        ],
    )(x, V, wy, vectors, W2a, W2b, W2c, Wlin, P64, P48)


def _perm64():
    p = np.zeros((4 * MUL, 4 * MUL), np.float32)
    for c in range(MUL):
        for i in range(4):
            p[4 * c + i, MUL * i + c] = 1.0
    return jnp.asarray(p)


def _perm48():
    p = np.zeros((3 * MUL, 3 * MUL), np.float32)
    for i in range(3):
        for o in range(MUL):
            p[MUL * i + o, 3 * o + i] = 1.0
    return jnp.asarray(p)


SUB = 125
ROWS = N_EDGES // SUB
NS = 16
NC = 2
RPT = ROWS // NS
CH = 5
NCHUNK = RPT // CH
NZ = N_NODES // NS


def _sc_body(msg_hbm, snd_hbm, zeros_hbm, wy_hbm, idx_v, data_v, table_sh):
    cid = lax.axis_index("c")
    tid = lax.axis_index("s")
    base = tid * RPT
    pltpu.sync_copy(zeros_hbm, table_sh.at[pl.ds(tid * NZ, NZ)])
    plsc.subcore_barrier()

    def scat(k, carry):
        r0 = base + k * CH
        pltpu.sync_copy(snd_hbm.at[pl.ds(r0, CH)], idx_v)
        pltpu.sync_copy(msg_hbm.at[cid, pl.ds(r0, CH)], data_v)
        for j in range(CH):
            pltpu.sync_copy(data_v.at[j], table_sh.at[idx_v.at[j]], add=True)
        return carry

    lax.fori_loop(0, NCHUNK, scat, 0)
    plsc.subcore_barrier()

    def gath(k, carry):
        r0 = base + k * CH
        pltpu.sync_copy(snd_hbm.at[pl.ds(r0, CH)], idx_v)
        for j in range(CH):
            pltpu.sync_copy(table_sh.at[idx_v.at[j]], data_v.at[j])
        pltpu.sync_copy(data_v, wy_hbm.at[cid, pl.ds(r0, CH)])
        return carry

    lax.fori_loop(0, NCHUNK, gath, 0)


def _segment_mid(msg, senders):
    msg_r = msg.reshape(NC, ROWS, SUB, 2 * MUL)
    snd_r = senders.reshape(ROWS, SUB)
    zeros = jnp.zeros((NZ, 2 * MUL), jnp.float32)
    mesh = plsc.VectorSubcoreMesh(core_axis_name="c", subcore_axis_name="s",
                                  num_cores=NC, num_subcores=NS)
    wy = pl.kernel(
        _sc_body,
        out_type=jax.ShapeDtypeStruct((NC, ROWS, SUB, 2 * MUL), jnp.float32),
        mesh=mesh,
        compiler_params=pltpu.CompilerParams(use_tc_tiling_on_sc=False),
        scratch_types=[
            pltpu.VMEM((CH, SUB), jnp.int32),
            pltpu.VMEM((CH, SUB, 2 * MUL), jnp.float32),
            pltpu.VMEM_SHARED((N_NODES, 2 * MUL), jnp.float32),
        ],
    )(msg_r, snd_r, zeros)
    return wy.reshape(NC, N_EDGES, 2 * MUL)


def kernel(vectors, x, V, senders, species, W1, W2a, W2b, W2c, Wlin):
    msg = _pass1(x, vectors, W1)
    wy = _segment_mid(msg, senders)
    x_out, v_out = _pass2(x, V, wy, vectors, W2a, W2b, W2c, Wlin,
                          _perm64(), _perm48())
    return x_out, v_out


# speedup vs baseline: 51.2249x; 2.7386x over previous
import jax
import jax.numpy as jnp
import numpy as np
from jax import lax
from jax.experimental import pallas as pl
from jax.experimental.pallas import tpu as pltpu
from jax.experimental.pallas import tpu_sc as plsc

N_NODES = 50000
N_EDGES = 800000
MUL = 16
X_DIM = 64
HIDDEN = 64
EPS = 0.25
SQRT3 = 1.7320508075688772

BE = 6400


def _mats():
    m = {}
    def w1q(W1):
        return jnp.concatenate([W1] * 4, axis=1) * (EPS / 8.0)
    m['w1q'] = w1q
    b3 = np.zeros((3, 4 * MUL), np.float32)
    for i in range(3):
        b3[i, MUL * (i + 1):MUL * (i + 2)] = SQRT3
    m['b3'] = b3
    m['ones3'] = np.ones((3, 4 * MUL), np.float32)
    d = np.zeros((4 * MUL, 4 * MUL), np.float32)
    for c in range(MUL):
        for i in range(4):
            d[c, MUL * i + c] = 1.0
    p64 = np.zeros((4 * MUL, 4 * MUL), np.float32)
    for c in range(MUL):
        for i in range(4):
            p64[4 * c + i, MUL * i + c] = 1.0
    eye = np.eye(4 * MUL, dtype=np.float32)
    m['cA'] = np.concatenate([p64, p64 @ d], axis=1)
    m['cB'] = np.concatenate([d, eye], axis=1)
    scat = []
    for i in range(1, 4):
        s_i = np.zeros((MUL, 3 * MUL), np.float32)
        for o in range(MUL):
            s_i[o, 3 * o + (i - 1)] = 1.0
        scat.append(s_i)
    m['lscat'] = scat

    def lmat(Wlin):
        s = 1.0 / np.sqrt(np.float32(2 * MUL))
        wl0 = Wlin[0:MUL] * s
        wl1 = Wlin[MUL:2 * MUL] * s
        z = jnp.zeros((MUL, 3 * MUL), jnp.float32)
        rows = [z] + [wl0 @ s_i for s_i in m['lscat']]
        rows += [z] + [wl1 @ s_i for s_i in m['lscat']]
        return jnp.concatenate(rows, axis=0)
    m['lmat'] = lmat
    r = np.zeros((4 * MUL, 2 * MUL), np.float32)
    for c in range(MUL):
        r[c, c] = 1.0
        for i in range(1, 4):
            r[MUL * i + c, MUL + c] = 1.0 / SQRT3
    m['r'] = r
    return m


_M = _mats()


def _mm_l(a, w):
    return lax.dot_general(a, w, (((0,), (0,)), ((), ())),
                           preferred_element_type=jnp.float32)


def _mm_r(w, a):
    return lax.dot_general(w, a, (((1,), (1,)), ((), ())),
                           preferred_element_type=jnp.float32)


def _pass1_body(xt_ref, vect_ref, w1q_ref, b3_ref, msg_ref):
    xt = xt_ref[...]
    vt = vect_ref[...]
    n2 = jnp.sum(vt * vt, axis=0, keepdims=True)
    r = 1.0 / jnp.maximum(jnp.sqrt(n2), 1e-9)
    ut = vt * r
    wmat = _mm_l(xt, w1q_ref[...])
    ue = _mm_l(ut, b3_ref[...])
    lane = lax.broadcasted_iota(jnp.int32, (BE, 4 * MUL), 1)
    one0 = jnp.where(lane < MUL, 1.0, 0.0)
    msg_ref[...] = wmat * (ue + one0)


def _pass1(xt, vect, W1):
    grid = N_EDGES // BE
    return pl.pallas_call(
        _pass1_body,
        grid=(grid,),
        in_specs=[
            pl.BlockSpec((X_DIM, BE), lambda i: (0, i)),
            pl.BlockSpec((3, BE), lambda i: (0, i)),
            pl.BlockSpec((X_DIM, 4 * MUL), lambda i: (0, 0)),
            pl.BlockSpec((3, 4 * MUL), lambda i: (0, 0)),
        ],
        out_specs=[pl.BlockSpec((BE, 4 * MUL), lambda i: (i, 0))],
        out_shape=[jax.ShapeDtypeStruct((N_EDGES, 4 * MUL), jnp.float32)],
    )(xt, vect, _M['w1q'](W1), _M['b3'])[0]


def _pass2_body(xt_ref, vT_ref, wy_ref, vect_ref, ca_ref, cb_ref, lt_ref,
                w2axt_ref, w2apt_ref, w2bt_ref, w2ct_ref,
                xout_ref, vout_ref):
    wy = wy_ref[...]
    vT = vT_ref[...]
    a = _mm_l(vT, ca_ref[...])
    b = jnp.dot(wy, cb_ref[...], preferred_element_type=jnp.float32)
    c2 = a * b
    vout_ref[...] = _mm_r(lt_ref[...], c2)
    p = wy * a[:, 0:4 * MUL]
    xt = xt_ref[...]
    pre = (jnp.dot(w2axt_ref[...], xt, preferred_element_type=jnp.float32)
           + _mm_r(w2apt_ref[...], p))
    h = pre * jax.nn.sigmoid(pre)
    h2 = jnp.dot(w2bt_ref[...], h, preferred_element_type=jnp.float32)
    h2 = h2 * jax.nn.sigmoid(h2)
    h3 = jnp.dot(w2ct_ref[...], h2, preferred_element_type=jnp.float32)
    vt = vect_ref[...]
    n2 = jnp.sum(vt * vt, axis=0, keepdims=True)
    t = n2 * n2 * n2
    u = jnp.sqrt(n2)
    env = jnp.where(n2 < 1.0, 1.0 - t * (28.0 - 48.0 * u + 21.0 * n2), 0.0)
    xout_ref[...] = env * h3


def _pass2(xt, VT, wy, vect, W2a, W2b, W2c, Wlin):
    s96 = 1.0 / np.sqrt(np.float32(X_DIM + 2 * MUL))
    w2axt = (W2a[0:X_DIM] * s96).T
    w2apt = ((_M['r'] @ W2a[X_DIM:]) * s96).T
    w2bt = (W2b * (1.0 / 8.0)).T
    w2ct = (W2c * (1.0 / 8.0)).T
    lt = _M['lmat'](Wlin).T
    grid = N_EDGES // BE
    return pl.pallas_call(
        _pass2_body,
        grid=(grid,),
        in_specs=[
            pl.BlockSpec((X_DIM, BE), lambda i: (0, i)),
            pl.BlockSpec((4 * MUL, BE), lambda i: (0, i)),
            pl.BlockSpec((BE, 4 * MUL), lambda i: (i, 0)),
            pl.BlockSpec((3, BE), lambda i: (0, i)),
            pl.BlockSpec((4 * MUL, 8 * MUL), lambda i: (0, 0)),
            pl.BlockSpec((4 * MUL, 8 * MUL), lambda i: (0, 0)),
            pl.BlockSpec((3 * MUL, 8 * MUL), lambda i: (0, 0)),
            pl.BlockSpec((HIDDEN, X_DIM), lambda i: (0, 0)),
            pl.BlockSpec((HIDDEN, 4 * MUL), lambda i: (0, 0)),
            pl.BlockSpec((HIDDEN, HIDDEN), lambda i: (0, 0)),
            pl.BlockSpec((HIDDEN, HIDDEN), lambda i: (0, 0)),
        ],
        out_specs=[
            pl.BlockSpec((HIDDEN, BE), lambda i: (0, i)),
            pl.BlockSpec((3 * MUL, BE), lambda i: (0, i)),
        ],
        out_shape=[
            jax.ShapeDtypeStruct((HIDDEN, N_EDGES), jnp.float32),
            jax.ShapeDtypeStruct((3 * MUL, N_EDGES), jnp.float32),
        ],
    )(xt, VT, wy, vect, _M['cA'], _M['cB'], lt,
      w2axt, w2apt, w2bt, w2ct)


SUB = 125
ROWS = N_EDGES // SUB
NS = 16
NC = 2
RPT = ROWS // NS
CH = 5
NCHUNK = RPT // CH
NZ = N_NODES // NS


def _sc_body(msg_hbm, snd_hbm, zeros_hbm, wy_hbm, idx_v, data_v, table_sh):
    cid = lax.axis_index("c")
    tid = lax.axis_index("s")
    base = tid * RPT
    lane0 = cid * (2 * MUL)
    pltpu.sync_copy(zeros_hbm, table_sh.at[pl.ds(tid * NZ, NZ)])
    plsc.subcore_barrier()

    def scat(k, carry):
        r0 = base + k * CH
        pltpu.sync_copy(snd_hbm.at[pl.ds(r0, CH)], idx_v)
        pltpu.sync_copy(msg_hbm.at[pl.ds(r0, CH), :, pl.ds(lane0, 2 * MUL)],
                        data_v)
        for j in range(CH):
            pltpu.sync_copy(data_v.at[j], table_sh.at[idx_v.at[j]], add=True)
        return carry

    lax.fori_loop(0, NCHUNK, scat, 0)
    plsc.subcore_barrier()

    def gath(k, carry):
        r0 = base + k * CH
        pltpu.sync_copy(snd_hbm.at[pl.ds(r0, CH)], idx_v)
        for j in range(CH):
            pltpu.sync_copy(table_sh.at[idx_v.at[j]], data_v.at[j])
        pltpu.sync_copy(data_v,
                        wy_hbm.at[pl.ds(r0, CH), :, pl.ds(lane0, 2 * MUL)])
        return carry

    lax.fori_loop(0, NCHUNK, gath, 0)


def _segment_mid(msg, senders):
    msg_r = msg.reshape(ROWS, SUB, 4 * MUL)
    snd_r = senders.reshape(ROWS, SUB)
    zeros = jnp.zeros((NZ, 2 * MUL), jnp.float32)
    mesh = plsc.VectorSubcoreMesh(core_axis_name="c", subcore_axis_name="s",
                                  num_cores=NC, num_subcores=NS)
    wy = pl.kernel(
        _sc_body,
        out_type=jax.ShapeDtypeStruct((ROWS, SUB, 4 * MUL), jnp.float32),
        mesh=mesh,
        compiler_params=pltpu.CompilerParams(use_tc_tiling_on_sc=False),
        scratch_types=[
            pltpu.VMEM((CH, SUB), jnp.int32),
            pltpu.VMEM((CH, SUB, 2 * MUL), jnp.float32),
            pltpu.VMEM_SHARED((N_NODES, 2 * MUL), jnp.float32),
        ],
    )(msg_r, snd_r, zeros)
    return wy.reshape(N_EDGES, 4 * MUL)


def kernel(vectors, x, V, senders, species, W1, W2a, W2b, W2c, Wlin):
    xt = x.T
    VT = V.T
    vect = vectors.T
    msg = _pass1(xt, vect, W1)
    wy = _segment_mid(msg, senders)
    xout_t, vout_t = _pass2(xt, VT, wy, vect, W2a, W2b, W2c, Wlin)
    return xout_t.T, vout_t.T


# speedup vs baseline: 57.1521x; 1.1157x over previous
import jax
import jax.numpy as jnp
import numpy as np
from jax import lax
from jax.experimental import pallas as pl
from jax.experimental.pallas import tpu as pltpu
from jax.experimental.pallas import tpu_sc as plsc

N_NODES = 50000
N_EDGES = 800000
MUL = 16
X_DIM = 64
HIDDEN = 64
EPS = 0.25
SQRT3 = 1.7320508075688772

BE = 6400


def _mats():
    m = {}
    def w1q(W1):
        return jnp.concatenate([W1] * 4, axis=1) * (EPS / 8.0)
    m['w1q'] = w1q
    b3 = np.zeros((3, 4 * MUL), np.float32)
    for i in range(3):
        b3[i, MUL * (i + 1):MUL * (i + 2)] = SQRT3
    m['b3'] = b3
    m['ones3'] = np.ones((3, 4 * MUL), np.float32)
    d = np.zeros((4 * MUL, 4 * MUL), np.float32)
    for c in range(MUL):
        for i in range(4):
            d[c, MUL * i + c] = 1.0
    p64 = np.zeros((4 * MUL, 4 * MUL), np.float32)
    for c in range(MUL):
        for i in range(4):
            p64[4 * c + i, MUL * i + c] = 1.0
    eye = np.eye(4 * MUL, dtype=np.float32)
    m['cA'] = np.concatenate([p64, p64 @ d], axis=1)
    m['cB'] = np.concatenate([d, eye], axis=1)
    scat = []
    for i in range(1, 4):
        s_i = np.zeros((MUL, 3 * MUL), np.float32)
        for o in range(MUL):
            s_i[o, 3 * o + (i - 1)] = 1.0
        scat.append(s_i)
    m['lscat'] = scat

    def lmat(Wlin):
        s = 1.0 / np.sqrt(np.float32(2 * MUL))
        wl0 = Wlin[0:MUL] * s
        wl1 = Wlin[MUL:2 * MUL] * s
        z = jnp.zeros((MUL, 3 * MUL), jnp.float32)
        rows = [z] + [wl0 @ s_i for s_i in m['lscat']]
        rows += [z] + [wl1 @ s_i for s_i in m['lscat']]
        return jnp.concatenate(rows, axis=0)
    m['lmat'] = lmat
    r = np.zeros((4 * MUL, 2 * MUL), np.float32)
    for c in range(MUL):
        r[c, c] = 1.0
        for i in range(1, 4):
            r[MUL * i + c, MUL + c] = 1.0 / SQRT3
    m['r'] = r
    return m


_M = _mats()


def _mm_l(a, w):
    return lax.dot_general(a, w, (((0,), (0,)), ((), ())),
                           preferred_element_type=jnp.float32)


def _mm_r(w, a):
    return lax.dot_general(w, a, (((1,), (1,)), ((), ())),
                           preferred_element_type=jnp.float32)


def _pass1_body(xt_ref, vect_ref, w1q_ref, b3_ref, msg_ref):
    xt = xt_ref[...]
    vt = vect_ref[...]
    n2 = jnp.sum(vt * vt, axis=0, keepdims=True)
    r = 1.0 / jnp.maximum(jnp.sqrt(n2), 1e-9)
    ut = vt * r
    wmat = _mm_l(xt, w1q_ref[...])
    ue = _mm_l(ut, b3_ref[...])
    lane = lax.broadcasted_iota(jnp.int32, (BE, 4 * MUL), 1)
    one0 = jnp.where(lane < MUL, 1.0, 0.0)
    msg_ref[...] = wmat * (ue + one0)


def _pass1(xt, vect, W1):
    grid = N_EDGES // BE
    return pl.pallas_call(
        _pass1_body,
        grid=(grid,),
        in_specs=[
            pl.BlockSpec((X_DIM, BE), lambda i: (0, i)),
            pl.BlockSpec((3, BE), lambda i: (0, i)),
            pl.BlockSpec((X_DIM, 4 * MUL), lambda i: (0, 0)),
            pl.BlockSpec((3, 4 * MUL), lambda i: (0, 0)),
        ],
        out_specs=[pl.BlockSpec((BE, 4 * MUL), lambda i: (i, 0))],
        out_shape=[jax.ShapeDtypeStruct((N_EDGES, 4 * MUL), jnp.float32)],
    )(xt, vect, _M['w1q'](W1), _M['b3'])[0]


def _pass2_body(xt_ref, vT_ref, wy_ref, vect_ref, ca_ref, cb_ref, lt_ref,
                w2axt_ref, w2apt_ref, w2bt_ref, w2ct_ref,
                xout_ref, vout_ref):
    wy = wy_ref[...]
    vT = vT_ref[...]
    a = _mm_l(vT, ca_ref[...])
    b = jnp.dot(wy, cb_ref[...], preferred_element_type=jnp.float32)
    c2 = a * b
    vout_ref[...] = _mm_r(lt_ref[...], c2)
    p = wy * a[:, 0:4 * MUL]
    xt = xt_ref[...]
    pre = (jnp.dot(w2axt_ref[...], xt, preferred_element_type=jnp.float32)
           + _mm_r(w2apt_ref[...], p))
    h = pre * jax.nn.sigmoid(pre)
    h2 = jnp.dot(w2bt_ref[...], h, preferred_element_type=jnp.float32)
    h2 = h2 * jax.nn.sigmoid(h2)
    h3 = jnp.dot(w2ct_ref[...], h2, preferred_element_type=jnp.float32)
    vt = vect_ref[...]
    n2 = jnp.sum(vt * vt, axis=0, keepdims=True)
    t = n2 * n2 * n2
    u = jnp.sqrt(n2)
    env = jnp.where(n2 < 1.0, 1.0 - t * (28.0 - 48.0 * u + 21.0 * n2), 0.0)
    xout_ref[...] = env * h3


def _pass2(xt, VT, wy, vect, W2a, W2b, W2c, Wlin):
    s96 = 1.0 / np.sqrt(np.float32(X_DIM + 2 * MUL))
    w2axt = (W2a[0:X_DIM] * s96).T
    w2apt = ((_M['r'] @ W2a[X_DIM:]) * s96).T
    w2bt = (W2b * (1.0 / 8.0)).T
    w2ct = (W2c * (1.0 / 8.0)).T
    lt = _M['lmat'](Wlin).T
    grid = N_EDGES // BE
    return pl.pallas_call(
        _pass2_body,
        grid=(grid,),
        in_specs=[
            pl.BlockSpec((X_DIM, BE), lambda i: (0, i)),
            pl.BlockSpec((4 * MUL, BE), lambda i: (0, i)),
            pl.BlockSpec((BE, 4 * MUL), lambda i: (i, 0)),
            pl.BlockSpec((3, BE), lambda i: (0, i)),
            pl.BlockSpec((4 * MUL, 8 * MUL), lambda i: (0, 0)),
            pl.BlockSpec((4 * MUL, 8 * MUL), lambda i: (0, 0)),
            pl.BlockSpec((3 * MUL, 8 * MUL), lambda i: (0, 0)),
            pl.BlockSpec((HIDDEN, X_DIM), lambda i: (0, 0)),
            pl.BlockSpec((HIDDEN, 4 * MUL), lambda i: (0, 0)),
            pl.BlockSpec((HIDDEN, HIDDEN), lambda i: (0, 0)),
            pl.BlockSpec((HIDDEN, HIDDEN), lambda i: (0, 0)),
        ],
        out_specs=[
            pl.BlockSpec((HIDDEN, BE), lambda i: (0, i)),
            pl.BlockSpec((3 * MUL, BE), lambda i: (0, i)),
        ],
        out_shape=[
            jax.ShapeDtypeStruct((HIDDEN, N_EDGES), jnp.float32),
            jax.ShapeDtypeStruct((3 * MUL, N_EDGES), jnp.float32),
        ],
    )(xt, VT, wy, vect, _M['cA'], _M['cB'], lt,
      w2axt, w2apt, w2bt, w2ct)


SUB = 125
ROWS = N_EDGES // SUB
NS = 16
NC = 2
RPT = ROWS // NS
CH = 2
NCHUNK = RPT // CH
NZ = N_NODES // NS


def _sc_body(msg_hbm, snd_hbm, zeros_hbm, wy_hbm, idx_v, data_v, table_sh,
             li0, li1, ld0, ld1, st0, st1):
    cid = lax.axis_index("c")
    tid = lax.axis_index("s")
    base = tid * RPT
    lane0 = cid * (2 * MUL)
    lsem = (li0, li1)
    dsem = (ld0, ld1)
    ssem = (st0, st1)

    def idx_dst(s):
        return idx_v.at[pl.ds(2 * s, CH)]

    def dat_dst(s):
        return data_v.at[pl.ds(2 * s, CH)]

    def snd_src(k):
        return snd_hbm.at[pl.ds(base + k * CH, CH)]

    def msg_src(k):
        return msg_hbm.at[pl.ds(base + k * CH, CH), :, pl.ds(lane0, 2 * MUL)]

    def wy_dst(k):
        return wy_hbm.at[pl.ds(base + k * CH, CH), :, pl.ds(lane0, 2 * MUL)]

    pltpu.sync_copy(zeros_hbm, table_sh.at[pl.ds(tid * NZ, NZ)])
    plsc.subcore_barrier()

    for s in range(2):
        pltpu.async_copy(snd_src(s), idx_dst(s), lsem[s])
        pltpu.async_copy(msg_src(s), dat_dst(s), dsem[s])

    def scat(i, carry):
        for s in range(2):
            k = 2 * i + s
            pltpu.make_async_copy(snd_src(k), idx_dst(s), lsem[s]).wait()
            pltpu.make_async_copy(msg_src(k), dat_dst(s), dsem[s]).wait()
            for j in range(CH):
                pltpu.sync_copy(data_v.at[2 * s + j],
                                table_sh.at[idx_v.at[2 * s + j]], add=True)

            @pl.when(k + 2 < NCHUNK)
            def _():
                pltpu.async_copy(snd_src(k + 2), idx_dst(s), lsem[s])
                pltpu.async_copy(msg_src(k + 2), dat_dst(s), dsem[s])
        return carry

    lax.fori_loop(0, NCHUNK // 2, scat, 0)
    plsc.subcore_barrier()

    for s in range(2):
        pltpu.async_copy(snd_src(s), idx_dst(s), lsem[s])

    def gath(i, carry):
        for s in range(2):
            k = 2 * i + s
            pltpu.make_async_copy(snd_src(k), idx_dst(s), lsem[s]).wait()

            @pl.when(k >= 2)
            def _():
                pltpu.make_async_copy(dat_dst(s), wy_dst(k - 2),
                                      ssem[s]).wait()
            for j in range(CH):
                pltpu.sync_copy(table_sh.at[idx_v.at[2 * s + j]],
                                data_v.at[2 * s + j])
            pltpu.async_copy(dat_dst(s), wy_dst(k), ssem[s])

            @pl.when(k + 2 < NCHUNK)
            def _():
                pltpu.async_copy(snd_src(k + 2), idx_dst(s), lsem[s])
        return carry

    lax.fori_loop(0, NCHUNK // 2, gath, 0)
    pltpu.make_async_copy(dat_dst(0), wy_dst(NCHUNK - 2), st0).wait()
    pltpu.make_async_copy(dat_dst(1), wy_dst(NCHUNK - 1), st1).wait()


def _segment_mid(msg, senders):
    msg_r = msg.reshape(ROWS, SUB, 4 * MUL)
    snd_r = senders.reshape(ROWS, SUB)
    zeros = jnp.zeros((NZ, 2 * MUL), jnp.float32)
    mesh = plsc.VectorSubcoreMesh(core_axis_name="c", subcore_axis_name="s",
                                  num_cores=NC, num_subcores=NS)
    wy = pl.kernel(
        _sc_body,
        out_type=jax.ShapeDtypeStruct((ROWS, SUB, 4 * MUL), jnp.float32),
        mesh=mesh,
        compiler_params=pltpu.CompilerParams(use_tc_tiling_on_sc=False),
        scratch_types=[
            pltpu.VMEM((2 * CH, SUB), jnp.int32),
            pltpu.VMEM((2 * CH, SUB, 2 * MUL), jnp.float32),
            pltpu.VMEM_SHARED((N_NODES, 2 * MUL), jnp.float32),
            pltpu.SemaphoreType.DMA,
            pltpu.SemaphoreType.DMA,
            pltpu.SemaphoreType.DMA,
            pltpu.SemaphoreType.DMA,
            pltpu.SemaphoreType.DMA,
            pltpu.SemaphoreType.DMA,
        ],
    )(msg_r, snd_r, zeros)
    return wy.reshape(N_EDGES, 4 * MUL)


def kernel(vectors, x, V, senders, species, W1, W2a, W2b, W2c, Wlin):
    xt = x.T
    VT = V.T
    vect = vectors.T
    msg = _pass1(xt, vect, W1)
    wy = _segment_mid(msg, senders)
    xout_t, vout_t = _pass2(xt, VT, wy, vect, W2a, W2b, W2c, Wlin)
    return xout_t.T, vout_t.T


# speedup vs baseline: 78.1606x; 1.3676x over previous
import jax
import jax.numpy as jnp
import numpy as np
from jax import lax
from jax.experimental import pallas as pl
from jax.experimental.pallas import tpu as pltpu
from jax.experimental.pallas import tpu_sc as plsc

N_NODES = 50000
N_EDGES = 800000
MUL = 16
X_DIM = 64
HIDDEN = 64
EPS = 0.25
SQRT3 = 1.7320508075688772

BE = 6400


def _mats():
    m = {}
    def w1q(W1):
        return jnp.concatenate([W1] * 4, axis=1) * (EPS / 8.0)
    m['w1q'] = w1q
    b3 = np.zeros((3, 4 * MUL), np.float32)
    for i in range(3):
        b3[i, MUL * (i + 1):MUL * (i + 2)] = SQRT3
    m['b3'] = b3
    m['ones3'] = np.ones((3, 4 * MUL), np.float32)
    d = np.zeros((4 * MUL, 4 * MUL), np.float32)
    for c in range(MUL):
        for i in range(4):
            d[c, MUL * i + c] = 1.0
    p64 = np.zeros((4 * MUL, 4 * MUL), np.float32)
    for c in range(MUL):
        for i in range(4):
            p64[4 * c + i, MUL * i + c] = 1.0
    eye = np.eye(4 * MUL, dtype=np.float32)
    m['cA'] = np.concatenate([p64, p64 @ d], axis=1)
    m['cB'] = np.concatenate([d, eye], axis=1)
    scat = []
    for i in range(1, 4):
        s_i = np.zeros((MUL, 3 * MUL), np.float32)
        for o in range(MUL):
            s_i[o, 3 * o + (i - 1)] = 1.0
        scat.append(s_i)
    m['lscat'] = scat

    def lmat(Wlin):
        s = 1.0 / np.sqrt(np.float32(2 * MUL))
        wl0 = Wlin[0:MUL] * s
        wl1 = Wlin[MUL:2 * MUL] * s
        z = jnp.zeros((MUL, 3 * MUL), jnp.float32)
        rows = [z] + [wl0 @ s_i for s_i in m['lscat']]
        rows += [z] + [wl1 @ s_i for s_i in m['lscat']]
        return jnp.concatenate(rows, axis=0)
    m['lmat'] = lmat
    r = np.zeros((4 * MUL, 2 * MUL), np.float32)
    for c in range(MUL):
        r[c, c] = 1.0
        for i in range(1, 4):
            r[MUL * i + c, MUL + c] = 1.0 / SQRT3
    m['r'] = r
    return m


_M = _mats()


def _mm_l(a, w):
    return lax.dot_general(a, w, (((0,), (0,)), ((), ())),
                           preferred_element_type=jnp.float32)


def _mm_r(w, a):
    return lax.dot_general(w, a, (((1,), (1,)), ((), ())),
                           preferred_element_type=jnp.float32)


def _pass1_body(xt_ref, vect_ref, w1q_ref, b3_ref, msg_ref):
    xt = xt_ref[...]
    vt = vect_ref[...]
    n2 = jnp.sum(vt * vt, axis=0, keepdims=True)
    r = 1.0 / jnp.maximum(jnp.sqrt(n2), 1e-9)
    ut = vt * r
    wmat = _mm_l(xt, w1q_ref[...])
    ue = _mm_l(ut, b3_ref[...])
    lane = lax.broadcasted_iota(jnp.int32, (BE, 4 * MUL), 1)
    one0 = jnp.where(lane < MUL, 1.0, 0.0)
    msg = wmat * (ue + one0)
    msg_ref[...] = jnp.concatenate([msg[0:BE // 2], msg[BE // 2:]], axis=1)


def _pass1(xt, vect, W1):
    grid = N_EDGES // BE
    return pl.pallas_call(
        _pass1_body,
        grid=(grid,),
        in_specs=[
            pl.BlockSpec((X_DIM, BE), lambda i: (0, i)),
            pl.BlockSpec((3, BE), lambda i: (0, i)),
            pl.BlockSpec((X_DIM, 4 * MUL), lambda i: (0, 0)),
            pl.BlockSpec((3, 4 * MUL), lambda i: (0, 0)),
        ],
        out_specs=[pl.BlockSpec((BE // 2, 8 * MUL), lambda i: (i, 0))],
        out_shape=[jax.ShapeDtypeStruct((N_EDGES // 2, 8 * MUL), jnp.float32)],
    )(xt, vect, _M['w1q'](W1), _M['b3'])[0]


def _pass2_body(xt_ref, vT_ref, wy_ref, vect_ref, ca_ref, cb_ref, lt_ref,
                w2axt_ref, w2apt_ref, w2bt_ref, w2ct_ref,
                xout_ref, vout_ref):
    wy2 = wy_ref[...]
    wy = jnp.concatenate([wy2[:, 0:4 * MUL], wy2[:, 4 * MUL:]], axis=0)
    vT = vT_ref[...]
    a = _mm_l(vT, ca_ref[...])
    b = jnp.dot(wy, cb_ref[...], preferred_element_type=jnp.float32)
    c2 = a * b
    vout_ref[...] = _mm_r(lt_ref[...], c2)
    p = wy * a[:, 0:4 * MUL]
    xt = xt_ref[...]
    pre = (jnp.dot(w2axt_ref[...], xt, preferred_element_type=jnp.float32)
           + _mm_r(w2apt_ref[...], p))
    h = pre * jax.nn.sigmoid(pre)
    h2 = jnp.dot(w2bt_ref[...], h, preferred_element_type=jnp.float32)
    h2 = h2 * jax.nn.sigmoid(h2)
    h3 = jnp.dot(w2ct_ref[...], h2, preferred_element_type=jnp.float32)
    vt = vect_ref[...]
    n2 = jnp.sum(vt * vt, axis=0, keepdims=True)
    t = n2 * n2 * n2
    u = jnp.sqrt(n2)
    env = jnp.where(n2 < 1.0, 1.0 - t * (28.0 - 48.0 * u + 21.0 * n2), 0.0)
    xout_ref[...] = env * h3


def _pass2(xt, VT, wy, vect, W2a, W2b, W2c, Wlin):
    s96 = 1.0 / np.sqrt(np.float32(X_DIM + 2 * MUL))
    w2axt = (W2a[0:X_DIM] * s96).T
    w2apt = ((_M['r'] @ W2a[X_DIM:]) * s96).T
    w2bt = (W2b * (1.0 / 8.0)).T
    w2ct = (W2c * (1.0 / 8.0)).T
    lt = _M['lmat'](Wlin).T
    grid = N_EDGES // BE
    return pl.pallas_call(
        _pass2_body,
        grid=(grid,),
        in_specs=[
            pl.BlockSpec((X_DIM, BE), lambda i: (0, i)),
            pl.BlockSpec((4 * MUL, BE), lambda i: (0, i)),
            pl.BlockSpec((BE // 2, 8 * MUL), lambda i: (i, 0)),
            pl.BlockSpec((3, BE), lambda i: (0, i)),
            pl.BlockSpec((4 * MUL, 8 * MUL), lambda i: (0, 0)),
            pl.BlockSpec((4 * MUL, 8 * MUL), lambda i: (0, 0)),
            pl.BlockSpec((3 * MUL, 8 * MUL), lambda i: (0, 0)),
            pl.BlockSpec((HIDDEN, X_DIM), lambda i: (0, 0)),
            pl.BlockSpec((HIDDEN, 4 * MUL), lambda i: (0, 0)),
            pl.BlockSpec((HIDDEN, HIDDEN), lambda i: (0, 0)),
            pl.BlockSpec((HIDDEN, HIDDEN), lambda i: (0, 0)),
        ],
        out_specs=[
            pl.BlockSpec((HIDDEN, BE), lambda i: (0, i)),
            pl.BlockSpec((3 * MUL, BE), lambda i: (0, i)),
        ],
        out_shape=[
            jax.ShapeDtypeStruct((HIDDEN, N_EDGES), jnp.float32),
            jax.ShapeDtypeStruct((3 * MUL, N_EDGES), jnp.float32),
        ],
    )(xt, VT, wy, vect, _M['cA'], _M['cB'], lt,
      w2axt, w2apt, w2bt, w2ct)


SUB = 125
ROWS = N_EDGES // SUB
NS = 16
NC = 2
RPT = ROWS // NS
CH = 2
NCHUNK = RPT // CH
NZ = N_NODES // NS


def _sc_body(msg_hbm, snd_hbm, zeros_hbm, wy_hbm, idx_v, data_v, table_sh,
             li0, li1, ld0, ld1, st0, st1):
    cid = lax.axis_index("c")
    tid = lax.axis_index("s")
    base = tid * RPT
    lane0 = cid * (2 * MUL)
    lsem = (li0, li1)
    dsem = (ld0, ld1)
    ssem = (st0, st1)

    def idx_dst(s):
        return idx_v.at[pl.ds(2 * s, CH)]

    def dat_dst(s):
        return data_v.at[pl.ds(2 * s, CH)]

    def snd_src(k):
        return snd_hbm.at[pl.ds(base + k * CH, CH)]

    def msg_src(k):
        return msg_hbm.at[pl.ds(base + k * CH, CH), :, pl.ds(lane0, 2 * MUL)]

    def wy_dst(k):
        return wy_hbm.at[pl.ds(base + k * CH, CH), :, pl.ds(lane0, 2 * MUL)]

    pltpu.sync_copy(zeros_hbm, table_sh.at[pl.ds(tid * NZ, NZ)])
    plsc.subcore_barrier()

    for s in range(2):
        pltpu.async_copy(snd_src(s), idx_dst(s), lsem[s])
        pltpu.async_copy(msg_src(s), dat_dst(s), dsem[s])

    def scat(i, carry):
        for s in range(2):
            k = 2 * i + s
            pltpu.make_async_copy(snd_src(k), idx_dst(s), lsem[s]).wait()
            pltpu.make_async_copy(msg_src(k), dat_dst(s), dsem[s]).wait()
            for j in range(CH):
                pltpu.sync_copy(data_v.at[2 * s + j],
                                table_sh.at[idx_v.at[2 * s + j]], add=True)

            @pl.when(k + 2 < NCHUNK)
            def _():
                pltpu.async_copy(snd_src(k + 2), idx_dst(s), lsem[s])
                pltpu.async_copy(msg_src(k + 2), dat_dst(s), dsem[s])
        return carry

    lax.fori_loop(0, NCHUNK // 2, scat, 0)
    plsc.subcore_barrier()

    for s in range(2):
        pltpu.async_copy(snd_src(s), idx_dst(s), lsem[s])

    def gath(i, carry):
        for s in range(2):
            k = 2 * i + s
            pltpu.make_async_copy(snd_src(k), idx_dst(s), lsem[s]).wait()

            @pl.when(k >= 2)
            def _():
                pltpu.make_async_copy(dat_dst(s), wy_dst(k - 2),
                                      ssem[s]).wait()
            for j in range(CH):
                pltpu.sync_copy(table_sh.at[idx_v.at[2 * s + j]],
                                data_v.at[2 * s + j])
            pltpu.async_copy(dat_dst(s), wy_dst(k), ssem[s])

            @pl.when(k + 2 < NCHUNK)
            def _():
                pltpu.async_copy(snd_src(k + 2), idx_dst(s), lsem[s])
        return carry

    lax.fori_loop(0, NCHUNK // 2, gath, 0)
    pltpu.make_async_copy(dat_dst(0), wy_dst(NCHUNK - 2), st0).wait()
    pltpu.make_async_copy(dat_dst(1), wy_dst(NCHUNK - 1), st1).wait()


def _segment_mid(msg, senders):
    msg_r = msg.reshape(ROWS, SUB, 4 * MUL)
    snd_r = senders.reshape(ROWS, SUB)
    zeros = jnp.zeros((NZ, 2 * MUL), jnp.float32)
    mesh = plsc.VectorSubcoreMesh(core_axis_name="c", subcore_axis_name="s",
                                  num_cores=NC, num_subcores=NS)
    wy = pl.kernel(
        _sc_body,
        out_type=jax.ShapeDtypeStruct((ROWS, SUB, 4 * MUL), jnp.float32),
        mesh=mesh,
        compiler_params=pltpu.CompilerParams(use_tc_tiling_on_sc=False),
        scratch_types=[
            pltpu.VMEM((2 * CH, SUB), jnp.int32),
            pltpu.VMEM((2 * CH, SUB, 2 * MUL), jnp.float32),
            pltpu.VMEM_SHARED((N_NODES, 2 * MUL), jnp.float32),
            pltpu.SemaphoreType.DMA,
            pltpu.SemaphoreType.DMA,
            pltpu.SemaphoreType.DMA,
            pltpu.SemaphoreType.DMA,
            pltpu.SemaphoreType.DMA,
            pltpu.SemaphoreType.DMA,
        ],
    )(msg_r, snd_r, zeros)
    return wy.reshape(N_EDGES // 2, 8 * MUL)


def kernel(vectors, x, V, senders, species, W1, W2a, W2b, W2c, Wlin):
    xt = x.T
    VT = V.T
    vect = vectors.T
    sp = senders.reshape(-1, 2, BE // 2).transpose(0, 2, 1).reshape(-1)
    msg = _pass1(xt, vect, W1)
    wy = _segment_mid(msg, sp)
    xout_t, vout_t = _pass2(xt, VT, wy, vect, W2a, W2b, W2c, Wlin)
    return xout_t.T, vout_t.T


# speedup vs baseline: 91.0038x; 1.1643x over previous
import jax
import jax.numpy as jnp
import numpy as np
from jax import lax
from jax.experimental import pallas as pl
from jax.experimental.pallas import tpu as pltpu
from jax.experimental.pallas import tpu_sc as plsc

N_NODES = 50000
N_EDGES = 800000
MUL = 16
X_DIM = 64
HIDDEN = 64
EPS = 0.25
SQRT3 = 1.7320508075688772

BE = 6400


def _mats():
    m = {}
    def w1q(W1):
        return jnp.concatenate([W1] * 4, axis=1) * (EPS / 8.0)
    m['w1q'] = w1q
    b3 = np.zeros((3, 4 * MUL), np.float32)
    for i in range(3):
        b3[i, MUL * (i + 1):MUL * (i + 2)] = SQRT3
    m['b3'] = b3
    m['ones3'] = np.ones((3, 4 * MUL), np.float32)
    d = np.zeros((4 * MUL, 4 * MUL), np.float32)
    for c in range(MUL):
        for i in range(4):
            d[c, MUL * i + c] = 1.0
    p64 = np.zeros((4 * MUL, 4 * MUL), np.float32)
    for c in range(MUL):
        for i in range(4):
            p64[4 * c + i, MUL * i + c] = 1.0
    eye = np.eye(4 * MUL, dtype=np.float32)
    m['cA'] = np.concatenate([p64, p64 @ d], axis=1)
    m['cB'] = np.concatenate([d, eye], axis=1)
    scat = []
    for i in range(1, 4):
        s_i = np.zeros((MUL, 3 * MUL), np.float32)
        for o in range(MUL):
            s_i[o, 3 * o + (i - 1)] = 1.0
        scat.append(s_i)
    m['lscat'] = scat

    def lmat(Wlin):
        s = 1.0 / np.sqrt(np.float32(2 * MUL))
        wl0 = Wlin[0:MUL] * s
        wl1 = Wlin[MUL:2 * MUL] * s
        z = jnp.zeros((MUL, 3 * MUL), jnp.float32)
        rows = [z] + [wl0 @ s_i for s_i in m['lscat']]
        rows += [z] + [wl1 @ s_i for s_i in m['lscat']]
        return jnp.concatenate(rows, axis=0)
    m['lmat'] = lmat
    r = np.zeros((4 * MUL, 2 * MUL), np.float32)
    for c in range(MUL):
        r[c, c] = 1.0
        for i in range(1, 4):
            r[MUL * i + c, MUL + c] = 1.0 / SQRT3
    m['r'] = r
    return m


_M = _mats()


def _mm_l(a, w):
    return lax.dot_general(a, w, (((0,), (0,)), ((), ())),
                           preferred_element_type=jnp.float32)


def _mm_r(w, a):
    return lax.dot_general(w, a, (((1,), (1,)), ((), ())),
                           preferred_element_type=jnp.float32)


def _pass1_body(xt_ref, vect_ref, w1q_ref, b3_ref, msg_ref):
    xt = xt_ref[...]
    vt = vect_ref[...]
    n2 = jnp.sum(vt * vt, axis=0, keepdims=True)
    r = 1.0 / jnp.maximum(jnp.sqrt(n2), 1e-9)
    ut = vt * r
    wmat = _mm_l(xt, w1q_ref[...])
    ue = _mm_l(ut, b3_ref[...])
    lane = lax.broadcasted_iota(jnp.int32, (BE, 4 * MUL), 1)
    one0 = jnp.where(lane < MUL, 1.0, 0.0)
    msg = wmat * (ue + one0)
    msg_ref[...] = jnp.concatenate([msg[0:BE // 2], msg[BE // 2:]], axis=1)


def _pass1(xt, vect, W1):
    grid = N_EDGES // BE
    return pl.pallas_call(
        _pass1_body,
        grid=(grid,),
        in_specs=[
            pl.BlockSpec((X_DIM, BE), lambda i: (0, i)),
            pl.BlockSpec((3, BE), lambda i: (0, i)),
            pl.BlockSpec((X_DIM, 4 * MUL), lambda i: (0, 0)),
            pl.BlockSpec((3, 4 * MUL), lambda i: (0, 0)),
        ],
        out_specs=[pl.BlockSpec((BE // 2, 8 * MUL), lambda i: (i, 0))],
        out_shape=[jax.ShapeDtypeStruct((N_EDGES // 2, 8 * MUL), jnp.float32)],
    )(xt, vect, _M['w1q'](W1), _M['b3'])[0]


def _pass2_body(xt_ref, vT_ref, wy_ref, vect_ref, ca_ref, cb_ref, lt_ref,
                w2axt_ref, w2apt_ref, w2bt_ref, w2ct_ref,
                xout_ref, vout_ref):
    wy2 = wy_ref[...]
    wy = jnp.concatenate([wy2[:, 0:4 * MUL], wy2[:, 4 * MUL:]], axis=0)
    vT = vT_ref[...]
    a = _mm_l(vT, ca_ref[...])
    b = jnp.dot(wy, cb_ref[...], preferred_element_type=jnp.float32)
    c2 = a * b
    vout_ref[...] = _mm_r(lt_ref[...], c2)
    p = wy * a[:, 0:4 * MUL]
    xt = xt_ref[...]
    pre = (jnp.dot(w2axt_ref[...], xt, preferred_element_type=jnp.float32)
           + _mm_r(w2apt_ref[...], p))
    h = pre * jax.nn.sigmoid(pre)
    h2 = jnp.dot(w2bt_ref[...], h, preferred_element_type=jnp.float32)
    h2 = h2 * jax.nn.sigmoid(h2)
    h3 = jnp.dot(w2ct_ref[...], h2, preferred_element_type=jnp.float32)
    vt = vect_ref[...]
    n2 = jnp.sum(vt * vt, axis=0, keepdims=True)
    t = n2 * n2 * n2
    u = jnp.sqrt(n2)
    env = jnp.where(n2 < 1.0, 1.0 - t * (28.0 - 48.0 * u + 21.0 * n2), 0.0)
    xout_ref[...] = env * h3


def _pass2(xt, VT, wy, vect, W2a, W2b, W2c, Wlin):
    s96 = 1.0 / np.sqrt(np.float32(X_DIM + 2 * MUL))
    w2axt = (W2a[0:X_DIM] * s96).T
    w2apt = ((_M['r'] @ W2a[X_DIM:]) * s96).T
    w2bt = (W2b * (1.0 / 8.0)).T
    w2ct = (W2c * (1.0 / 8.0)).T
    lt = _M['lmat'](Wlin).T
    grid = N_EDGES // BE
    return pl.pallas_call(
        _pass2_body,
        grid=(grid,),
        in_specs=[
            pl.BlockSpec((X_DIM, BE), lambda i: (0, i)),
            pl.BlockSpec((4 * MUL, BE), lambda i: (0, i)),
            pl.BlockSpec((BE // 2, 8 * MUL), lambda i: (i, 0)),
            pl.BlockSpec((3, BE), lambda i: (0, i)),
            pl.BlockSpec((4 * MUL, 8 * MUL), lambda i: (0, 0)),
            pl.BlockSpec((4 * MUL, 8 * MUL), lambda i: (0, 0)),
            pl.BlockSpec((3 * MUL, 8 * MUL), lambda i: (0, 0)),
            pl.BlockSpec((HIDDEN, X_DIM), lambda i: (0, 0)),
            pl.BlockSpec((HIDDEN, 4 * MUL), lambda i: (0, 0)),
            pl.BlockSpec((HIDDEN, HIDDEN), lambda i: (0, 0)),
            pl.BlockSpec((HIDDEN, HIDDEN), lambda i: (0, 0)),
        ],
        out_specs=[
            pl.BlockSpec((HIDDEN, BE), lambda i: (0, i)),
            pl.BlockSpec((3 * MUL, BE), lambda i: (0, i)),
        ],
        out_shape=[
            jax.ShapeDtypeStruct((HIDDEN, N_EDGES), jnp.float32),
            jax.ShapeDtypeStruct((3 * MUL, N_EDGES), jnp.float32),
        ],
    )(xt, VT, wy, vect, _M['cA'], _M['cB'], lt,
      w2axt, w2apt, w2bt, w2ct)


SUB = 125
ROWS = N_EDGES // SUB
NS = 16
NC = 2
RPT = ROWS // NS
CH = 2
NCHUNK = RPT // CH
NZ = N_NODES // NS


def _sc_body(msg_hbm, snd_hbm, zeros_hbm, wy_hbm, idx_v, data_v, table_sh,
             li0, li1, ld0, ld1, st0, st1):
    cid = lax.axis_index("c")
    tid = lax.axis_index("s")
    base = tid * RPT
    lane0 = cid * (2 * MUL)
    lsem = (li0, li1)
    dsem = (ld0, ld1)
    ssem = (st0, st1)

    def idx_dst(s):
        return idx_v.at[pl.ds(2 * s, CH)]

    def dat_dst(s):
        return data_v.at[pl.ds(2 * s, CH)]

    def snd_src(k):
        return snd_hbm.at[pl.ds(base + k * CH, CH)]

    def msg_src(k):
        return msg_hbm.at[pl.ds(base + k * CH, CH), :, pl.ds(lane0, 2 * MUL)]

    def wy_dst(k):
        return wy_hbm.at[pl.ds(base + k * CH, CH), :, pl.ds(lane0, 2 * MUL)]

    pltpu.sync_copy(zeros_hbm, table_sh.at[pl.ds(tid * NZ, NZ)])
    plsc.subcore_barrier()

    for s in range(2):
        pltpu.async_copy(snd_src(s), idx_dst(s), lsem[s])
        pltpu.async_copy(msg_src(s), dat_dst(s), dsem[s])

    def scat(i, carry):
        for s in range(2):
            k = 2 * i + s
            pltpu.make_async_copy(snd_src(k), idx_dst(s), lsem[s]).wait()
            pltpu.make_async_copy(msg_src(k), dat_dst(s), dsem[s]).wait()
            for j in range(CH):
                pltpu.sync_copy(data_v.at[2 * s + j],
                                table_sh.at[idx_v.at[2 * s + j]], add=True)

            @pl.when(k + 2 < NCHUNK)
            def _():
                pltpu.async_copy(snd_src(k + 2), idx_dst(s), lsem[s])
                pltpu.async_copy(msg_src(k + 2), dat_dst(s), dsem[s])
        return carry

    lax.fori_loop(0, NCHUNK // 2, scat, 0)
    plsc.subcore_barrier()

    for s in range(2):
        pltpu.async_copy(snd_src(s), idx_dst(s), lsem[s])

    def gath(i, carry):
        for s in range(2):
            k = 2 * i + s
            pltpu.make_async_copy(snd_src(k), idx_dst(s), lsem[s]).wait()

            @pl.when(k >= 2)
            def _():
                pltpu.make_async_copy(dat_dst(s), wy_dst(k - 2),
                                      ssem[s]).wait()
            for j in range(CH):
                pltpu.sync_copy(table_sh.at[idx_v.at[2 * s + j]],
                                data_v.at[2 * s + j])
            pltpu.async_copy(dat_dst(s), wy_dst(k), ssem[s])

            @pl.when(k + 2 < NCHUNK)
            def _():
                pltpu.async_copy(snd_src(k + 2), idx_dst(s), lsem[s])
        return carry

    lax.fori_loop(0, NCHUNK // 2, gath, 0)
    pltpu.make_async_copy(dat_dst(0), wy_dst(NCHUNK - 2), st0).wait()
    pltpu.make_async_copy(dat_dst(1), wy_dst(NCHUNK - 1), st1).wait()


def _segment_mid(msg, senders):
    msg_r = msg.reshape(ROWS, SUB, 4 * MUL)
    snd_r = senders.reshape(ROWS, SUB)
    zeros = jnp.zeros((NZ, 2 * MUL), jnp.float32)
    mesh = plsc.VectorSubcoreMesh(core_axis_name="c", subcore_axis_name="s",
                                  num_cores=NC, num_subcores=NS)
    wy = pl.kernel(
        _sc_body,
        out_type=jax.ShapeDtypeStruct((ROWS, SUB, 4 * MUL), jnp.float32),
        mesh=mesh,
        compiler_params=pltpu.CompilerParams(use_tc_tiling_on_sc=False),
        scratch_types=[
            pltpu.VMEM((2 * CH, SUB), jnp.int32),
            pltpu.VMEM((2 * CH, SUB, 2 * MUL), jnp.float32),
            pltpu.VMEM_SHARED((N_NODES, 2 * MUL), jnp.float32),
            pltpu.SemaphoreType.DMA,
            pltpu.SemaphoreType.DMA,
            pltpu.SemaphoreType.DMA,
            pltpu.SemaphoreType.DMA,
            pltpu.SemaphoreType.DMA,
            pltpu.SemaphoreType.DMA,
        ],
    )(msg_r, snd_r, zeros)
    return wy.reshape(N_EDGES // 2, 8 * MUL)


def kernel(vectors, x, V, senders, species, W1, W2a, W2b, W2c, Wlin):
    xt = x.T
    VT = V.T
    vect = vectors.T
    perm = np.arange(N_EDGES).reshape(-1, 2, BE // 2)
    perm = perm.transpose(0, 2, 1).reshape(-1).astype(np.int32)
    sp = jnp.take(senders, jnp.asarray(perm), axis=0)
    msg = _pass1(xt, vect, W1)
    wy = _segment_mid(msg, sp)
    xout_t, vout_t = _pass2(xt, VT, wy, vect, W2a, W2b, W2c, Wlin)
    return xout_t.T, vout_t.T
